# Initial kernel scaffold; baseline (speedup 1.0000x reference)
import jax
import jax.numpy as jnp
from jax.experimental import pallas as pl


def kernel(local_coordinate, flattened_index, features):
    raise NotImplementedError("write your pallas kernel here")


# speedup vs baseline: 6.3881x; 6.3881x over previous
import jax
import jax.numpy as jnp
from jax import lax
from jax.experimental import pallas as pl
from jax.experimental.pallas import tpu as pltpu
from jax.experimental.pallas import tpu_sc as plsc

HEADS = 4
FD = 16
TOTAL = 32768
B = 2
V = 8
P = 16384

NC, NS = 2, 16
NW = NC * NS
PLANES = B * HEADS * FD
ELEMS = V * P
CHUNK = 4096
NCHUNK = ELEMS // CHUNK
GROUPS = CHUNK // 16


def _scatter_max(tbl, idxv, val):
    cur = plsc.load_gather(tbl, [idxv])
    new = jnp.maximum(cur, val)
    plsc.store_scatter(tbl, [idxv], new, mask=val > cur)
    chk = plsc.load_gather(tbl, [idxv])

    @pl.when(jnp.any(chk < new))
    def _():
        def body(i, c):
            plsc.store_scatter(tbl, [idxv], new, mask=c < new)
            return plsc.load_gather(tbl, [idxv])

        lax.fori_loop(0, 15, body, chk)


def _splat_body(coord_hbm, idx_hbm, feats_hbm, out_hbm,
                t0, t1, fr0, fr1, ib, cb):
    wid = lax.axis_index("s") * NC + lax.axis_index("c")

    for pj in range(2):
        plane0 = wid * 4 + pj * 2
        plane1 = plane0 + 1
        grp = plane0 // 16
        ebase = grp * ELEMS

        zv = jnp.zeros((16,), jnp.float32)

        def zbody(i, _):
            t0[pl.ds(i * 16, 16)] = zv
            t1[pl.ds(i * 16, 16)] = zv
            return 0

        lax.fori_loop(0, TOTAL // 16, zbody, 0)

        pltpu.sync_copy(feats_hbm.at[pl.ds(plane0 * P, P)], fr0)
        pltpu.sync_copy(feats_hbm.at[pl.ds(plane1 * P, P)], fr1)

        def chunk_body(ch, _):
            start = ebase + ch * CHUNK
            pltpu.sync_copy(idx_hbm.at[pl.ds(start, CHUNK)], ib)
            pltpu.sync_copy(coord_hbm.at[pl.ds(start, CHUNK)], cb)
            p0 = lax.rem(ch, P // CHUNK) * CHUNK

            def gbody(g, _):
                s = g * 16
                idxv = ib[pl.ds(s, 16)]
                crd = cb[pl.ds(s, 16)]
                val0 = fr0[pl.ds(p0 + s, 16)] * crd
                val1 = fr1[pl.ds(p0 + s, 16)] * crd
                _scatter_max(t0, idxv, val0)
                _scatter_max(t1, idxv, val1)
                return 0

            lax.fori_loop(0, GROUPS, gbody, 0)
            return 0

        lax.fori_loop(0, NCHUNK, chunk_body, 0)

        pltpu.sync_copy(t0, out_hbm.at[pl.ds(plane0 * TOTAL, TOTAL)])
        pltpu.sync_copy(t1, out_hbm.at[pl.ds(plane1 * TOTAL, TOTAL)])


def kernel(local_coordinate, flattened_index, features):
    coord_flat = local_coordinate.reshape(-1)
    idx_flat = flattened_index.reshape(-1)
    feats_flat = features.reshape(-1)

    mesh = plsc.VectorSubcoreMesh(
        core_axis_name="c", subcore_axis_name="s",
        num_cores=NC, num_subcores=NS,
    )
    out = pl.kernel(
        _splat_body,
        out_type=jax.ShapeDtypeStruct((PLANES * TOTAL,), jnp.float32),
        mesh=mesh,
        scratch_types=[
            pltpu.VMEM((TOTAL,), jnp.float32),
            pltpu.VMEM((TOTAL,), jnp.float32),
            pltpu.VMEM((P,), jnp.float32),
            pltpu.VMEM((P,), jnp.float32),
            pltpu.VMEM((CHUNK,), jnp.int32),
            pltpu.VMEM((CHUNK,), jnp.float32),
        ],
        compiler_params=pltpu.CompilerParams(needs_layout_passes=False),
        name="splat_scatter_max",
    )(coord_flat, idx_flat, feats_flat)

    return out.reshape(B, HEADS * FD, 32, 32, 32)


# speedup vs baseline: 74.6688x; 11.6888x over previous
import jax
import jax.numpy as jnp
from jax import lax
from jax.experimental import pallas as pl
from jax.experimental.pallas import tpu as pltpu
from jax.experimental.pallas import tpu_sc as plsc

HEADS = 4
FD = 16
TOTAL = 32768
B = 2
V = 8
P = 16384

NC, NS = 2, 16
NW = NC * NS
PLANES = B * HEADS * FD
ELEMS = V * P
CHUNK = 4096
NCHUNK = ELEMS // CHUNK
GROUPS = CHUNK // 16


def _splat_body(coord_hbm, idx_hbm, feats_hbm, out_hbm,
                t0, t1, fr0, fr1, ib0, cb0, ib1, cb1,
                sem_i0, sem_c0, sem_i1, sem_c1, sem_f):
    wid = lax.axis_index("s") * NC + lax.axis_index("c")

    def resolve_dups(tbl, idxv, val):
        def body(i, c):
            plsc.store_scatter(tbl, [idxv], jnp.maximum(c, val),
                               mask=val > c)
            return plsc.load_gather(tbl, [idxv])

        lax.fori_loop(0, 16, body, plsc.load_gather(tbl, [idxv]))

    def fix_pair(iA, iB, a0, a1, b0, b1, viol):
        @pl.when(viol)
        def _():
            resolve_dups(t0, iA, a0)
            resolve_dups(t1, iA, a1)
            resolve_dups(t0, iB, b0)
            resolve_dups(t1, iB, b1)

    def process(ib, cb, p0):
        def gbody(g, carry):
            piA, piB, pa0, pa1, pb0, pb1, badp = carry
            violp = jnp.any(badp)
            s = g * 32
            iA = ib[pl.ds(s, 16)]
            iB = ib[pl.ds(s + 16, 16)]
            cA = cb[pl.ds(s, 16)]
            cB = cb[pl.ds(s + 16, 16)]
            a0 = fr0[pl.ds(p0 + s, 16)] * cA
            a1 = fr1[pl.ds(p0 + s, 16)] * cA
            b0 = fr0[pl.ds(p0 + s + 16, 16)] * cB
            b1 = fr1[pl.ds(p0 + s + 16, 16)] * cB
            curA0 = plsc.load_gather(t0, [iA])
            curB0 = plsc.load_gather(t0, [iB])
            curA1 = plsc.load_gather(t1, [iA])
            curB1 = plsc.load_gather(t1, [iB])
            nA0 = jnp.maximum(curA0, a0)
            nB0 = jnp.maximum(curB0, b0)
            nA1 = jnp.maximum(curA1, a1)
            nB1 = jnp.maximum(curB1, b1)
            plsc.store_scatter(t0, [iA], nA0, mask=a0 > curA0)
            plsc.store_scatter(t0, [iB], nB0, mask=b0 > curB0)
            plsc.store_scatter(t1, [iA], nA1, mask=a1 > curA1)
            plsc.store_scatter(t1, [iB], nB1, mask=b1 > curB1)
            fix_pair(piA, piB, pa0, pa1, pb0, pb1, violp)
            chkA0 = plsc.load_gather(t0, [iA])
            chkB0 = plsc.load_gather(t0, [iB])
            chkA1 = plsc.load_gather(t1, [iA])
            chkB1 = plsc.load_gather(t1, [iB])
            bad = ((nA0 > chkA0) | (nB0 > chkB0)
                   | (nA1 > chkA1) | (nB1 > chkB1))
            return iA, iB, a0, a1, b0, b1, bad

        def gbody2(g, carry):
            carry = gbody(g * 2, carry)
            return gbody(g * 2 + 1, carry)

        z_i = jnp.zeros((16,), jnp.int32)
        z_f = jnp.zeros((16,), jnp.float32)
        carry = (z_i, z_i, z_f, z_f, z_f, z_f,
                 jnp.zeros((16,), jnp.bool_))
        iA, iB, a0, a1, b0, b1, bad = lax.fori_loop(
            0, GROUPS // 4, gbody2, carry)
        fix_pair(iA, iB, a0, a1, b0, b1, jnp.any(bad))

    for pj in range(2):
        plane0 = wid * 4 + pj * 2
        plane1 = plane0 + 1
        grp = plane0 // 16
        ebase = grp * ELEMS

        f_cp0 = pltpu.async_copy(feats_hbm.at[pl.ds(plane0 * P, P)], fr0,
                                 sem_f)
        f_cp1 = pltpu.async_copy(feats_hbm.at[pl.ds(plane1 * P, P)], fr1,
                                 sem_f)
        pltpu.async_copy(idx_hbm.at[pl.ds(ebase, CHUNK)], ib0, sem_i0)
        pltpu.async_copy(coord_hbm.at[pl.ds(ebase, CHUNK)], cb0, sem_c0)

        zv = jnp.zeros((16,), jnp.float32)

        def zbody(i, _):
            s = i * 64
            for u in range(4):
                t0[pl.ds(s + u * 16, 16)] = zv
                t1[pl.ds(s + u * 16, 16)] = zv
            return 0

        lax.fori_loop(0, TOTAL // 64, zbody, 0)
        f_cp0.wait()
        f_cp1.wait()

        def chunk_pair(j, _):
            c0 = j * 2
            st1 = ebase + (c0 + 1) * CHUNK
            pltpu.async_copy(idx_hbm.at[pl.ds(st1, CHUNK)], ib1, sem_i1)
            pltpu.async_copy(coord_hbm.at[pl.ds(st1, CHUNK)], cb1, sem_c1)
            pltpu.make_async_copy(idx_hbm.at[pl.ds(st1, CHUNK)], ib0,
                                  sem_i0).wait()
            pltpu.make_async_copy(coord_hbm.at[pl.ds(st1, CHUNK)], cb0,
                                  sem_c0).wait()
            process(ib0, cb0, lax.rem(c0, P // CHUNK) * CHUNK)
            nxt = jnp.minimum(c0 + 2, NCHUNK - 1)
            st0 = ebase + nxt * CHUNK
            pltpu.async_copy(idx_hbm.at[pl.ds(st0, CHUNK)], ib0, sem_i0)
            pltpu.async_copy(coord_hbm.at[pl.ds(st0, CHUNK)], cb0, sem_c0)
            pltpu.make_async_copy(idx_hbm.at[pl.ds(st1, CHUNK)], ib1,
                                  sem_i1).wait()
            pltpu.make_async_copy(coord_hbm.at[pl.ds(st1, CHUNK)], cb1,
                                  sem_c1).wait()
            process(ib1, cb1, lax.rem(c0 + 1, P // CHUNK) * CHUNK)
            return 0

        lax.fori_loop(0, NCHUNK // 2, chunk_pair, 0)
        pltpu.make_async_copy(idx_hbm.at[pl.ds(ebase, CHUNK)], ib0,
                              sem_i0).wait()
        pltpu.make_async_copy(coord_hbm.at[pl.ds(ebase, CHUNK)], cb0,
                              sem_c0).wait()

        pltpu.sync_copy(t0, out_hbm.at[pl.ds(plane0 * TOTAL, TOTAL)])
        pltpu.sync_copy(t1, out_hbm.at[pl.ds(plane1 * TOTAL, TOTAL)])


def kernel(local_coordinate, flattened_index, features):
    coord_flat = local_coordinate.reshape(-1)
    idx_flat = flattened_index.reshape(-1)
    feats_flat = features.reshape(-1)

    mesh = plsc.VectorSubcoreMesh(
        core_axis_name="c", subcore_axis_name="s",
        num_cores=NC, num_subcores=NS,
    )
    out = pl.kernel(
        _splat_body,
        out_type=jax.ShapeDtypeStruct((PLANES * TOTAL,), jnp.float32),
        mesh=mesh,
        scratch_types=[
            pltpu.VMEM((TOTAL,), jnp.float32),
            pltpu.VMEM((TOTAL,), jnp.float32),
            pltpu.VMEM((P,), jnp.float32),
            pltpu.VMEM((P,), jnp.float32),
            pltpu.VMEM((CHUNK,), jnp.int32),
            pltpu.VMEM((CHUNK,), jnp.float32),
            pltpu.VMEM((CHUNK,), jnp.int32),
            pltpu.VMEM((CHUNK,), jnp.float32),
            pltpu.SemaphoreType.DMA,
            pltpu.SemaphoreType.DMA,
            pltpu.SemaphoreType.DMA,
            pltpu.SemaphoreType.DMA,
            pltpu.SemaphoreType.DMA,
        ],
        compiler_params=pltpu.CompilerParams(needs_layout_passes=False),
        name="splat_scatter_max",
    )(coord_flat, idx_flat, feats_flat)

    return out.reshape(B, HEADS * FD, 32, 32, 32)
